# baseline (speedup 1.0000x reference)
import functools

import jax
import jax.numpy as jnp
from jax import lax
from jax.experimental import pallas as pl
from jax.experimental.pallas import tpu as pltpu
from jax.experimental.pallas import tpu_sc as plsc

NUM_FEATURES = 4
CHUNKS_PER_FEATURE = 2
NUM_TABLES = 8
BUCKETS = 100000
DIM = 32
BATCH = 16384

PBLK = 512
PGRID = (BUCKETS + PBLK - 1) // PBLK
PROWS = PGRID * 128
VROWS = PROWS * 4
BLOCKS_PER_WORKER = NUM_TABLES * PGRID // 32

NUM_WORKERS = 32
WORKERS_PER_CHUNK = NUM_WORKERS // NUM_TABLES
ROWS_PER_WORKER = BATCH // WORKERS_PER_CHUNK
BLK = 1024
NBLK = ROWS_PER_WORKER // BLK
SUB = 128
NSUB = BLK // SUB
LANES = 16


NFULL = NUM_TABLES * (PGRID - 1)


def _repack_body(tab_hbm, out_hbm, x_v, y_v):
    wid = lax.axis_index("s") * 2 + lax.axis_index("c")
    iota = lax.broadcasted_iota(jnp.int32, (16,), 0)

    def transpose_block(width):
        def do_chunk(m, _):
            cq = m // 4
            dhi = m % 4
            colbase = cq * DIM + dhi * 8
            for dlo in range(8):
                col = jnp.full((16,), colbase + dlo, jnp.int32)
                for g in range(8):
                    v = x_v[dhi * 8 + dlo, pl.dslice(cq * 128 + g * 16, 16)]
                    plsc.store_scatter(y_v, [g * 16 + iota, col], v)
            return 0

        lax.fori_loop(0, (width // 128) * 4, do_chunk, 0)

    def do_block(blockid, _):
        t = blockid // (PGRID - 1)
        j = blockid % (PGRID - 1)
        b0 = pl.multiple_of(j * PBLK, 128)
        pltpu.sync_copy(tab_hbm.at[t, :, pl.dslice(b0, PBLK)], x_v)
        transpose_block(PBLK)
        pltpu.sync_copy(
            y_v.at[:, pl.dslice(0, 128)],
            out_hbm.at[t, pl.dslice(pl.multiple_of(j * 128, 128), 128), :])
        return 0

    start = (NFULL * wid) // NUM_WORKERS
    end = (NFULL * (wid + 1)) // NUM_WORKERS
    lax.fori_loop(start, end, do_block, 0)

    @pl.when(wid < NUM_TABLES)
    def _():
        t = wid
        b0 = pl.multiple_of((PGRID - 1) * PBLK, 128)
        pltpu.sync_copy(tab_hbm.at[t, :, pl.dslice(b0, 256)],
                        x_v.at[:, pl.dslice(0, 256)])
        transpose_block(256)
        pltpu.sync_copy(
            y_v.at[:, pl.dslice(0, 128)],
            out_hbm.at[t, pl.dslice(pl.multiple_of((PGRID - 1) * 128, 128),
                                    128), :])


def _sc_repack(tab_t):
    mesh = plsc.VectorSubcoreMesh(core_axis_name="c", subcore_axis_name="s")
    run = functools.partial(
        pl.kernel,
        out_type=jax.ShapeDtypeStruct((NUM_TABLES, PROWS, 128), jnp.float32),
        mesh=mesh,
        scratch_types=[
            pltpu.VMEM((DIM, PBLK), jnp.float32),
            pltpu.VMEM((128, 136), jnp.float32),
        ],
        compiler_params=pltpu.CompilerParams(needs_layout_passes=False),
    )(_repack_body)
    return run(tab_t)


def _gather_body(tab_hbm, feats_hbm, out_hbm, feat_v, idx_v, rows_v, sem):
    wid = lax.axis_index("s") * 2 + lax.axis_index("c")
    chunk = wid // WORKERS_PER_CHUNK
    quarter = wid % WORKERS_PER_CHUNK
    f = chunk // CHUNKS_PER_FEATURE
    c = chunk % CHUNKS_PER_FEATURE

    f_u = f.astype(jnp.uint32)
    c_u = c.astype(jnp.uint32)
    mult0 = jnp.uint32(2654435761) + jnp.uint32(2) * f_u + jnp.uint32(1)
    add0 = c_u * jnp.uint32(40503) + jnp.uint32(97)
    tab_base = chunk.astype(jnp.uint32) * jnp.uint32(VROWS)

    row_base = quarter * ROWS_PER_WORKER
    feat_base = f * BATCH + row_base
    col0 = c * DIM

    def do_block(blk, _):
        row0 = blk * BLK
        pltpu.sync_copy(feats_hbm.at[pl.dslice(feat_base + row0, BLK)], feat_v)

        def hash_row(j, _):
            for ii in range(SUB // LANES):
                x = feat_v[pl.dslice(j * SUB + ii * LANES, LANES)]
                h = x.astype(jnp.uint32)
                h = h * mult0
                h = h + add0
                h = h ^ (h >> jnp.uint32(16))
                h = h * jnp.uint32(2246822519)
                h = h ^ (h >> jnp.uint32(13))
                h = h % jnp.uint32(BUCKETS)
                h = (tab_base
                     + ((h >> jnp.uint32(9)) << jnp.uint32(9))
                     + ((h & jnp.uint32(127)) << jnp.uint32(2))
                     + ((h >> jnp.uint32(7)) & jnp.uint32(3)))
                idx_v[j, pl.dslice(ii * LANES, LANES)] = h.astype(jnp.int32)
            return 0

        lax.fori_loop(0, NSUB, hash_row, 0)

        copies = [
            pltpu.async_copy(
                tab_hbm.at[idx_v.at[j]],
                rows_v.at[pl.dslice(j * SUB, SUB)],
                sem,
            )
            for j in range(NSUB)
        ]
        for cp in copies:
            cp.wait()

        pltpu.sync_copy(
            rows_v,
            out_hbm.at[f, pl.dslice(row_base + row0, BLK), pl.dslice(col0, DIM)],
        )
        return 0

    lax.fori_loop(0, NBLK, do_block, 0)


def kernel(tables, feat_0, feat_1, feat_2, feat_3):
    tab32 = _sc_repack(tables.transpose(0, 2, 1)).reshape(
        NUM_TABLES * VROWS, DIM)

    feats = jnp.stack([feat_0, feat_1, feat_2, feat_3]).reshape(NUM_FEATURES * BATCH)

    mesh = plsc.VectorSubcoreMesh(core_axis_name="c", subcore_axis_name="s")
    run = functools.partial(
        pl.kernel,
        out_type=jax.ShapeDtypeStruct(
            (NUM_FEATURES, BATCH, CHUNKS_PER_FEATURE * DIM), jnp.float32),
        mesh=mesh,
        scratch_types=[
            pltpu.VMEM((BLK,), jnp.int32),
            pltpu.VMEM((NSUB, SUB), jnp.int32),
            pltpu.VMEM((BLK, DIM), jnp.float32),
            pltpu.SemaphoreType.DMA,
        ],
        compiler_params=pltpu.CompilerParams(use_tc_tiling_on_sc=False),
    )(_gather_body)

    return run(tab32, feats)


# speedup vs baseline: 1.5750x; 1.5750x over previous
import functools

import jax
import jax.numpy as jnp
from jax import lax
from jax.experimental import pallas as pl
from jax.experimental.pallas import tpu as pltpu
from jax.experimental.pallas import tpu_sc as plsc

NUM_FEATURES = 4
CHUNKS_PER_FEATURE = 2
NUM_TABLES = 8
BUCKETS = 100000
DIM = 32
BATCH = 16384

NUM_WORKERS = 32
WORKERS_PER_CHUNK = NUM_WORKERS // NUM_TABLES
ROWS_PER_WORKER = BATCH // WORKERS_PER_CHUNK
BLK = 1024
NBLK = ROWS_PER_WORKER // BLK
SUB = 128
NSUB = BLK // SUB
LANES = 16


def _body(tab_hbm, feats_hbm, out_hbm, feat_v, idx_v, rows_v, sem):
    wid = lax.axis_index("s") * 2 + lax.axis_index("c")
    chunk = wid // WORKERS_PER_CHUNK
    quarter = wid % WORKERS_PER_CHUNK
    f = chunk // CHUNKS_PER_FEATURE
    c = chunk % CHUNKS_PER_FEATURE

    f_u = f.astype(jnp.uint32)
    c_u = c.astype(jnp.uint32)
    mult0 = jnp.uint32(2654435761) + jnp.uint32(2) * f_u + jnp.uint32(1)
    add0 = c_u * jnp.uint32(40503) + jnp.uint32(97)

    row_base = quarter * ROWS_PER_WORKER
    feat_base = f * BATCH + row_base
    col0 = c * DIM

    def do_block(blk, _):
        row0 = blk * BLK
        pltpu.sync_copy(feats_hbm.at[pl.dslice(feat_base + row0, BLK)], feat_v)

        def hash_row(j, _):
            for ii in range(SUB // LANES):
                x = feat_v[pl.dslice(j * SUB + ii * LANES, LANES)]
                h = x.astype(jnp.uint32)
                h = h * mult0
                h = h + add0
                h = h ^ (h >> jnp.uint32(16))
                h = h * jnp.uint32(2246822519)
                h = h ^ (h >> jnp.uint32(13))
                h = h % jnp.uint32(BUCKETS)
                idx_v[j, pl.dslice(ii * LANES, LANES)] = h.astype(jnp.int32)
            return 0

        lax.fori_loop(0, NSUB, hash_row, 0)

        copies = [
            pltpu.async_copy(
                tab_hbm.at[chunk].at[idx_v.at[j]],
                rows_v.at[pl.dslice(j * SUB, SUB)],
                sem,
            )
            for j in range(NSUB)
        ]
        for cp in copies:
            cp.wait()

        pltpu.sync_copy(
            rows_v,
            out_hbm.at[f, pl.dslice(row_base + row0, BLK), pl.dslice(col0, DIM)],
        )
        return 0

    lax.fori_loop(0, NBLK, do_block, 0)


def kernel(tables, feat_0, feat_1, feat_2, feat_3):
    feats = jnp.stack([feat_0, feat_1, feat_2, feat_3]).reshape(NUM_FEATURES * BATCH)

    mesh = plsc.VectorSubcoreMesh(core_axis_name="c", subcore_axis_name="s")
    run = functools.partial(
        pl.kernel,
        out_type=jax.ShapeDtypeStruct(
            (NUM_FEATURES, BATCH, CHUNKS_PER_FEATURE * DIM), jnp.float32),
        mesh=mesh,
        scratch_types=[
            pltpu.VMEM((BLK,), jnp.int32),
            pltpu.VMEM((NSUB, SUB), jnp.int32),
            pltpu.VMEM((BLK, DIM), jnp.float32),
            pltpu.SemaphoreType.DMA,
        ],
        compiler_params=pltpu.CompilerParams(use_tc_tiling_on_sc=False),
    )(_body)

    return run(tables, feats)
